# Initial kernel scaffold; baseline (speedup 1.0000x reference)
import jax
import jax.numpy as jnp
from jax.experimental import pallas as pl


def kernel(x, W):
    raise NotImplementedError("write your pallas kernel here")


# speedup vs baseline: 1.1856x; 1.1856x over previous
import functools

import jax
import jax.numpy as jnp
from jax.experimental import pallas as pl

TOPK = 8
NEXP = 64


def _router_block(x_ref, w_ref, probs_ref, idx_ref):
    xb = x_ref[...]
    wb = w_ref[...]
    logits = jax.lax.dot_general(
        xb, wb, (((1,), (1,)), ((), ())), preferred_element_type=jnp.float32
    )

    m = jnp.max(logits, axis=1, keepdims=True)
    e = jnp.exp(logits - m)
    s = jnp.sum(e, axis=1, keepdims=True)
    p = e / s

    lane = jax.lax.broadcasted_iota(jnp.int32, p.shape, 1)
    vals = p
    top_v = []
    top_i = []
    for _ in range(TOPK):
        mv = jnp.max(vals, axis=1, keepdims=True)
        mi = jnp.min(jnp.where(vals == mv, lane, NEXP), axis=1, keepdims=True)
        top_v.append(mv)
        top_i.append(mi)
        vals = jnp.where(lane == mi, -1.0, vals)

    tv = jnp.concatenate(top_v, axis=1)
    ti = jnp.concatenate(top_i, axis=1)
    tv = tv / jnp.sum(tv, axis=1, keepdims=True)
    probs_ref[...] = tv
    idx_ref[...] = ti


@functools.partial(jax.jit, static_argnames=())
def kernel(x, W):
    B, T, D = x.shape
    N = B * T
    x_flat = x.reshape(N, D)
    R = 512
    grid = (N // R,)
    probs, idx = pl.pallas_call(
        _router_block,
        grid=grid,
        in_specs=[
            pl.BlockSpec((R, D), lambda i: (i, 0)),
            pl.BlockSpec((NEXP, D), lambda i: (0, 0)),
        ],
        out_specs=[
            pl.BlockSpec((R, TOPK), lambda i: (i, 0)),
            pl.BlockSpec((R, TOPK), lambda i: (i, 0)),
        ],
        out_shape=[
            jax.ShapeDtypeStruct((N, TOPK), jnp.float32),
            jax.ShapeDtypeStruct((N, TOPK), jnp.int32),
        ],
    )(x_flat, W)
    aux_loss = jnp.array(0.0, dtype=jnp.float32)
    return (probs, idx, aux_loss)


# speedup vs baseline: 1.4848x; 1.2524x over previous
import functools

import jax
import jax.numpy as jnp
from jax.experimental import pallas as pl

TOPK = 8
NEXP = 64


def _router_block(x_ref, w_ref, probs_ref, idx_ref):
    xb = x_ref[...]
    wb = w_ref[...]
    logits = jax.lax.dot_general(
        wb, xb, (((1,), (1,)), ((), ())), preferred_element_type=jnp.float32
    )

    m = jnp.max(logits, axis=0, keepdims=True)
    e = jnp.exp(logits - m)
    s = jnp.sum(e, axis=0, keepdims=True)
    p = e / s

    sub = jax.lax.broadcasted_iota(jnp.int32, p.shape, 0)
    vals = p
    top_v = []
    top_i = []
    for _ in range(TOPK):
        mv = jnp.max(vals, axis=0, keepdims=True)
        mi = jnp.min(jnp.where(vals == mv, sub, NEXP), axis=0, keepdims=True)
        top_v.append(mv)
        top_i.append(mi)
        vals = jnp.where(sub == mi, -1.0, vals)

    tv = jnp.concatenate(top_v, axis=0)
    ti = jnp.concatenate(top_i, axis=0)
    tv = tv / jnp.sum(tv, axis=0, keepdims=True)
    probs_ref[...] = tv.T
    idx_ref[...] = ti.T


@functools.partial(jax.jit, static_argnames=())
def kernel(x, W):
    B, T, D = x.shape
    N = B * T
    x_flat = x.reshape(N, D)
    R = 512
    grid = (N // R,)
    probs, idx = pl.pallas_call(
        _router_block,
        grid=grid,
        in_specs=[
            pl.BlockSpec((R, D), lambda i: (i, 0)),
            pl.BlockSpec((NEXP, D), lambda i: (0, 0)),
        ],
        out_specs=[
            pl.BlockSpec((R, TOPK), lambda i: (i, 0)),
            pl.BlockSpec((R, TOPK), lambda i: (i, 0)),
        ],
        out_shape=[
            jax.ShapeDtypeStruct((N, TOPK), jnp.float32),
            jax.ShapeDtypeStruct((N, TOPK), jnp.int32),
        ],
    )(x_flat, W)
    aux_loss = jnp.array(0.0, dtype=jnp.float32)
    return (probs, idx, aux_loss)


# speedup vs baseline: 1.6136x; 1.0867x over previous
import functools

import jax
import jax.numpy as jnp
from jax.experimental import pallas as pl

TOPK = 8
NEXP = 64


def _router_block(x_ref, w_ref, probs_ref, idx_ref):
    xb = x_ref[...]
    wb = w_ref[...]
    logits = jax.lax.dot_general(
        wb, xb, (((1,), (1,)), ((), ())), preferred_element_type=jnp.float32
    )

    m = jnp.max(logits, axis=0, keepdims=True)
    e = jnp.exp(logits - m)
    s = jnp.sum(e, axis=0, keepdims=True)
    p = e / s

    sub = jax.lax.broadcasted_iota(jnp.int32, p.shape, 0)
    vals = p
    top_v = []
    top_i = []
    for _ in range(TOPK):
        mv = jnp.max(vals, axis=0, keepdims=True)
        mi = jnp.min(jnp.where(vals == mv, sub, NEXP), axis=0, keepdims=True)
        top_v.append(mv)
        top_i.append(mi)
        vals = jnp.where(sub == mi, -1.0, vals)

    tv = jnp.concatenate(top_v, axis=0)
    ti = jnp.concatenate(top_i, axis=0)
    tv = tv / jnp.sum(tv, axis=0, keepdims=True)
    probs_ref[...] = tv.T
    idx_ref[...] = ti.T


@functools.partial(jax.jit, static_argnames=())
def kernel(x, W):
    B, T, D = x.shape
    N = B * T
    x_flat = x.reshape(N, D)
    R = 1024
    grid = (N // R,)
    probs, idx = pl.pallas_call(
        _router_block,
        grid=grid,
        in_specs=[
            pl.BlockSpec((R, D), lambda i: (i, 0)),
            pl.BlockSpec((NEXP, D), lambda i: (0, 0)),
        ],
        out_specs=[
            pl.BlockSpec((R, TOPK), lambda i: (i, 0)),
            pl.BlockSpec((R, TOPK), lambda i: (i, 0)),
        ],
        out_shape=[
            jax.ShapeDtypeStruct((N, TOPK), jnp.float32),
            jax.ShapeDtypeStruct((N, TOPK), jnp.int32),
        ],
    )(x_flat, W)
    aux_loss = jnp.array(0.0, dtype=jnp.float32)
    return (probs, idx, aux_loss)
